# baseline (speedup 1.0000x reference)
import functools

import jax
import jax.numpy as jnp
from jax import lax
from jax.experimental import pallas as pl
from jax.experimental.pallas import tpu as pltpu
from jax.experimental.pallas import tpu_sc as plsc

D = 64
LANES = 16
NUM_K = D // LANES


def _sc_partial_counts(x, weight, *, b_per_w, chunk):
    n_ch = b_per_w // chunk
    mesh = plsc.VectorSubcoreMesh(core_axis_name="c", subcore_axis_name="s")
    num_workers = 32

    @functools.partial(
        pl.kernel,
        mesh=mesh,
        compiler_params=pltpu.CompilerParams(use_tc_tiling_on_sc=False),
        out_type=jax.ShapeDtypeStruct((num_workers, D), jnp.float32),
        scratch_types=[
            pltpu.VMEM((b_per_w,), jnp.int32),
            pltpu.VMEM((2, chunk, D), jnp.float32),
            pltpu.VMEM((1, D), jnp.float32),
            pltpu.SemaphoreType.DMA,
            pltpu.SemaphoreType.DMA,
        ],
    )
    def body(x_hbm, w_hbm, out_hbm, idx_v, rows_v, acc_v, sem0, sem1):
        nc = 2
        wid = lax.axis_index("s") * nc + lax.axis_index("c")
        base = wid * b_per_w
        pltpu.sync_copy(x_hbm.at[pl.ds(base, b_per_w)], idx_v)

        sems = (sem0, sem1)

        def start(j):
            return pltpu.async_copy(
                w_hbm.at[idx_v.at[pl.ds(j * chunk, chunk)]],
                rows_v.at[j % 2],
                sems[j % 2],
            )

        accs = tuple(jnp.zeros((LANES,), jnp.float32) for _ in range(NUM_K))
        handles = [None] * n_ch
        handles[0] = start(0)
        for j in range(n_ch):
            if j + 1 < n_ch:
                handles[j + 1] = start(j + 1)
            handles[j].wait()
            buf = j % 2

            def row_body(i, accs, buf=buf):
                out = []
                for k in range(NUM_K):
                    v = rows_v[buf, i, pl.ds(k * LANES, LANES)]
                    out.append(accs[k] + jnp.where(v >= 0.0, 1.0, 0.0))
                return tuple(out)

            accs = lax.fori_loop(0, chunk, row_body, accs)

        for k in range(NUM_K):
            acc_v[0, pl.ds(k * LANES, LANES)] = accs[k]
        pltpu.sync_copy(acc_v, out_hbm.at[pl.ds(wid, 1)])

    return body(x, weight)


def _tc_combine(partials, threshold):
    def body(p_ref, o_ref):
        s = jnp.sum(p_ref[...], axis=0, keepdims=True)
        o_ref[...] = jnp.where(s >= threshold, 1.0, -1.0)

    return pl.pallas_call(
        body,
        out_shape=jax.ShapeDtypeStruct((1, D), jnp.float32),
    )(partials)


def kernel(x, _weight):
    batch = x.shape[0]
    partials = _sc_partial_counts(
        x.astype(jnp.int32), _weight, b_per_w=batch // 32, chunk=128
    )
    return _tc_combine(partials, float(batch) / 2.0)


# speedup vs baseline: 1.7171x; 1.7171x over previous
import functools

import jax
import jax.numpy as jnp
from jax import lax
from jax.experimental import pallas as pl
from jax.experimental.pallas import tpu as pltpu
from jax.experimental.pallas import tpu_sc as plsc

D = 64
LANES = 16
NUM_K = D // LANES


def _sc_partial_counts(x, weight, *, b_per_w, chunk):
    n_ch = b_per_w // chunk
    mesh = plsc.VectorSubcoreMesh(core_axis_name="c", subcore_axis_name="s")
    num_workers = 32

    @functools.partial(
        pl.kernel,
        mesh=mesh,
        out_type=jax.ShapeDtypeStruct((num_workers, D), jnp.float32),
        scratch_types=[
            pltpu.VMEM((b_per_w,), jnp.int32),
            pltpu.VMEM((2, chunk, D), jnp.float32),
            pltpu.VMEM((1, D), jnp.float32),
            pltpu.SemaphoreType.DMA,
            pltpu.SemaphoreType.DMA,
        ],
    )
    def body(x_hbm, w_hbm, out_hbm, idx_v, rows_v, acc_v, sem0, sem1):
        nc = 2
        wid = lax.axis_index("s") * nc + lax.axis_index("c")
        base = wid * b_per_w
        pltpu.sync_copy(x_hbm.at[pl.ds(base, b_per_w)], idx_v)

        sems = (sem0, sem1)

        def fire(c):
            buf = c % 2
            for g in range(chunk // LANES):
                vals = idx_v[pl.ds(c * chunk + g * LANES, LANES)]
                for j in range(LANES):
                    r = vals[j]
                    pltpu.async_copy(
                        w_hbm.at[pl.ds(r, 1)],
                        rows_v.at[buf, pl.ds(g * LANES + j, 1)],
                        sems[buf],
                    )

        def drain(c):
            buf = c % 2
            pltpu.make_async_copy(
                w_hbm.at[pl.ds(0, chunk)], rows_v.at[buf], sems[buf]
            ).wait()

        accs = tuple(jnp.zeros((LANES,), jnp.float32) for _ in range(NUM_K))
        fire(0)
        for c in range(n_ch):
            if c + 1 < n_ch:
                fire(c + 1)
            drain(c)
            buf = c % 2

            def row_body(i, accs, buf=buf):
                out = []
                for k in range(NUM_K):
                    v = rows_v[buf, i, pl.ds(k * LANES, LANES)]
                    out.append(accs[k] + jnp.where(v >= 0.0, 1.0, 0.0))
                return tuple(out)

            accs = lax.fori_loop(0, chunk, row_body, accs)

        for k in range(NUM_K):
            acc_v[0, pl.ds(k * LANES, LANES)] = accs[k]
        pltpu.sync_copy(acc_v, out_hbm.at[pl.ds(wid, 1)])

    return body(x, weight)


def _tc_combine(partials, threshold):
    def body(p_ref, o_ref):
        s = jnp.sum(p_ref[...], axis=0, keepdims=True)
        o_ref[...] = jnp.where(s >= threshold, 1.0, -1.0)

    return pl.pallas_call(
        body,
        out_shape=jax.ShapeDtypeStruct((1, D), jnp.float32),
    )(partials)


def kernel(x, _weight):
    batch = x.shape[0]
    partials = _sc_partial_counts(
        x.astype(jnp.int32), _weight, b_per_w=batch // 32, chunk=64
    )
    return _tc_combine(partials, float(batch) / 2.0)


# speedup vs baseline: 3.6732x; 2.1392x over previous
import functools

import jax
import jax.numpy as jnp
from jax import lax
from jax.experimental import pallas as pl
from jax.experimental.pallas import tpu as pltpu
from jax.experimental.pallas import tpu_sc as plsc

D = 64
LANES = 16
N_TILES = 16
BLK = 8192
N_BLK = 123
M_PAD = N_BLK * BLK
STRIPE = M_PAD // N_TILES
ZCH = STRIPE // 8


def _sc_histogram(x, *, b_per_w):
    n_idx_ch = b_per_w // 128
    mesh = plsc.VectorSubcoreMesh(core_axis_name="c", subcore_axis_name="s")

    @functools.partial(
        pl.kernel,
        mesh=mesh,
        out_type=jax.ShapeDtypeStruct((2, M_PAD), jnp.float32),
        scratch_types=[
            pltpu.VMEM_SHARED((M_PAD,), jnp.float32),
            pltpu.VMEM((n_idx_ch, 128), jnp.int32),
            pltpu.VMEM((128,), jnp.float32),
            pltpu.VMEM((ZCH,), jnp.float32),
        ],
    )
    def body(x_hbm, out_hbm, m_sp, idx_v, ones_v, zeros_v):
        cid = lax.axis_index("c")
        sid = lax.axis_index("s")
        base = (cid * N_TILES + sid) * b_per_w

        ones = jnp.ones((LANES,), jnp.float32)
        zeros = jnp.zeros((LANES,), jnp.float32)
        for g in range(128 // LANES):
            ones_v[pl.ds(g * LANES, LANES)] = ones

        def zfill(i, carry):
            zeros_v[pl.ds(i * LANES, LANES)] = zeros
            return carry

        lax.fori_loop(0, ZCH // LANES, zfill, 0)

        for k in range(n_idx_ch):
            pltpu.sync_copy(
                x_hbm.at[pl.ds(base + k * 128, 128)], idx_v.at[k]
            )

        for z in range(STRIPE // ZCH):
            pltpu.sync_copy(
                zeros_v, m_sp.at[pl.ds(sid * STRIPE + z * ZCH, ZCH)]
            )
        plsc.subcore_barrier()

        for k in range(n_idx_ch):
            pltpu.sync_copy(ones_v, m_sp.at[idx_v.at[k]], add=True)
        plsc.subcore_barrier()

        pltpu.sync_copy(
            m_sp.at[pl.ds(sid * STRIPE, STRIPE)],
            out_hbm.at[cid, pl.ds(sid * STRIPE, STRIPE)],
        )

    return body(x)


def _tc_scan(wt, m2, threshold):
    def body(w_ref, m_ref, o_ref, acc_ref):
        i = pl.program_id(0)
        msum = m_ref[0:1, :] + m_ref[1:2, :]
        t = jnp.where(w_ref[...] >= 0.0, msum, 0.0)

        @pl.when(i == 0)
        def _():
            acc_ref[...] = t

        @pl.when(i > 0)
        def _():
            acc_ref[...] += t

        @pl.when(i == pl.num_programs(0) - 1)
        def _():
            s = jnp.sum(acc_ref[...], axis=1)
            o_ref[...] = jnp.where(s >= threshold, 1.0, -1.0).reshape(1, D)

    return pl.pallas_call(
        body,
        grid=(N_BLK,),
        in_specs=[
            pl.BlockSpec((D, BLK), lambda i: (0, i)),
            pl.BlockSpec((2, BLK), lambda i: (0, i)),
        ],
        out_specs=pl.BlockSpec((1, D), lambda i: (0, 0)),
        out_shape=jax.ShapeDtypeStruct((1, D), jnp.float32),
        scratch_shapes=[pltpu.VMEM((D, BLK), jnp.float32)],
    )(wt, m2)


def kernel(x, _weight):
    batch = x.shape[0]
    wt = jnp.transpose(_weight)
    m2 = _sc_histogram(x.astype(jnp.int32), b_per_w=batch // 32)
    return _tc_scan(wt, m2, float(batch) / 2.0)
